# baseline (speedup 1.0000x reference)
import jax
import jax.numpy as jnp
from jax.experimental import pallas as pl
from jax.experimental.pallas import tpu as pltpu

V = 32000
PAD = 0
SMOOTHING = 0.1
CONF = 1.0 - SMOOTHING
EPS = SMOOTHING / (V - 2)

BR = 64
BC = 32000


def _row_losses(x, t):
    m = jnp.max(x, axis=1, keepdims=True)
    s = jnp.sum(jnp.exp(x - m), axis=1, keepdims=True)
    lane = jax.lax.broadcasted_iota(jnp.int32, x.shape, 1)
    wt = jnp.where(lane == t[:, None], CONF, EPS)
    w = jnp.sum(wt * x, axis=1, keepdims=True)
    w = w - EPS * x[:, 0:1]
    L = m + jnp.log(s)
    return jnp.where(t[:, None] == PAD, 0.0, L - w)


def _loss_kernel(ta_ref, tb_ref, xa_ref, xb_ref, oa_ref, ob_ref):
    oa_ref[...] = _row_losses(xa_ref[...], ta_ref[0, 0, :])
    ob_ref[...] = _row_losses(xb_ref[...], tb_ref[0, 0, :])


def kernel(pred, target):
    n = pred.shape[0] * pred.shape[1]
    p = pred.reshape(n, V)
    t = target.reshape(-1).astype(jnp.int32)
    half = n // 2
    nr = half // BR
    pa, pb = p[:half], p[half:]
    ta = t[:half].reshape(nr, 1, BR)
    tb = t[half:].reshape(nr, 1, BR)

    idx_t = lambda r: (r, 0, 0)
    idx_x = lambda r: (r, 0)
    la, lb = pl.pallas_call(
        _loss_kernel,
        grid=(nr,),
        in_specs=[
            pl.BlockSpec((1, 1, BR), idx_t),
            pl.BlockSpec((1, 1, BR), idx_t),
            pl.BlockSpec((BR, BC), idx_x),
            pl.BlockSpec((BR, BC), idx_x),
        ],
        out_specs=[
            pl.BlockSpec((BR, 1), idx_x),
            pl.BlockSpec((BR, 1), idx_x),
        ],
        out_shape=[
            jax.ShapeDtypeStruct((half, 1), jnp.float32),
            jax.ShapeDtypeStruct((half, 1), jnp.float32),
        ],
        compiler_params=pltpu.CompilerParams(
            dimension_semantics=("parallel",)),
    )(ta, tb, pa, pb)
    return (jnp.sum(la) + jnp.sum(lb)) / n
